# baseline (speedup 1.0000x reference)
import jax
import jax.numpy as jnp
from jax.experimental import pallas as pl
from jax.experimental.pallas import tpu as pltpu

_NORM = 1.0


def _decoder_kernel(a_hbm, h_ref, mask_ref,
                    wm0_ref, bm0_ref, wu0_ref, bu0_ref,
                    wm1_ref, bm1_ref, wu1_ref, bu1_ref,
                    wo_ref, bo_ref, out_ref, m_scr, a_vmem, sems):
    f32 = jnp.float32
    P = jax.lax.Precision.DEFAULT
    i = pl.program_id(0)
    num_b = pl.num_programs(0) // 2
    b = i // 2
    layer = jax.lax.rem(i, 2)
    slot = jax.lax.rem(b, 2)
    nslot = jax.lax.rem(b + 1, 2)

    n_chunks = 4
    rows = out_ref.shape[1] // n_chunks

    def _start_copies(src_b, dst_slot):
        for c in range(n_chunks):
            sl = pl.ds(c * rows, rows)
            pltpu.make_async_copy(a_hbm.at[src_b, sl],
                                  a_vmem.at[dst_slot, sl],
                                  sems.at[dst_slot, c]).start()

    @pl.when(i == 0)
    def _():
        _start_copies(0, 0)

    @pl.when((layer == 0) & (b + 1 < num_b))
    def _():
        _start_copies(b + 1, nslot)

    @pl.when(i == 0)
    def _():
        m_scr[...] = jnp.dot(h_ref[0], wm0_ref[...], precision=P,
                             preferred_element_type=f32) + bm0_ref[...]

    @pl.when(layer == 0)
    def _():
        for c in range(n_chunks):
            sl = pl.ds(c * rows, rows)
            pltpu.make_async_copy(a_hbm.at[b, sl], a_vmem.at[slot, sl],
                                  sems.at[slot, c]).wait()

    is_l0 = layer == 0
    wu = jnp.where(is_l0, wu0_ref[...], wu1_ref[...])
    bu = jnp.where(is_l0, bu0_ref[...], bu1_ref[...])

    agg = jnp.dot(a_vmem[slot], m_scr[...], precision=P,
                  preferred_element_type=f32)
    agg = agg * (1.0 / _NORM)
    h_next = jnp.maximum(
        jnp.dot(agg, wu, precision=P, preferred_element_type=f32) + bu, 0.0)

    @pl.when(layer == 0)
    def _():
        m_scr[...] = jnp.dot(h_next, wm1_ref[...], precision=P,
                             preferred_element_type=f32) + bm1_ref[...]

    @pl.when((layer == 1) & (b + 1 < num_b))
    def _():
        m_scr[...] = jnp.dot(h_ref[b + 1], wm0_ref[...], precision=P,
                             preferred_element_type=f32) + bm0_ref[...]

    @pl.when(layer == 1)
    def _():
        out = jnp.dot(h_next, wo_ref[...], precision=P,
                      preferred_element_type=f32) + bo_ref[...]
        out_ref[b] = out * mask_ref[b]


def kernel(latent_features, adjacency_matrix, node_mask,
           W_msg0, b_msg0, W_upd0, b_upd0,
           W_msg1, b_msg1, W_upd1, b_upd1,
           W_out, b_out):
    B, N, d_lat = latent_features.shape
    d_hid = W_msg0.shape[1]
    d_out = W_out.shape[1]

    b2 = lambda b: b.reshape(1, -1)

    vmem = pltpu.MemorySpace.VMEM

    return pl.pallas_call(
        _decoder_kernel,
        grid=(2 * B,),
        in_specs=[
            pl.BlockSpec(memory_space=pltpu.MemorySpace.HBM),
            pl.BlockSpec(memory_space=vmem),
            pl.BlockSpec(memory_space=vmem),
            pl.BlockSpec(memory_space=vmem), pl.BlockSpec(memory_space=vmem),
            pl.BlockSpec(memory_space=vmem), pl.BlockSpec(memory_space=vmem),
            pl.BlockSpec(memory_space=vmem), pl.BlockSpec(memory_space=vmem),
            pl.BlockSpec(memory_space=vmem), pl.BlockSpec(memory_space=vmem),
            pl.BlockSpec(memory_space=vmem), pl.BlockSpec(memory_space=vmem),
        ],
        out_specs=pl.BlockSpec(memory_space=vmem),
        out_shape=jax.ShapeDtypeStruct((B, N, d_out), jnp.float32),
        scratch_shapes=[
            pltpu.VMEM((N, d_hid), jnp.float32),
            pltpu.VMEM((2, N, N), jnp.float32),
            pltpu.SemaphoreType.DMA((2, 4)),
        ],
        compiler_params=pltpu.CompilerParams(
            dimension_semantics=("arbitrary",),
            vmem_limit_bytes=64 * 1024 * 1024,
        ),
    )(adjacency_matrix, latent_features, node_mask,
      W_msg0, b2(b_msg0), W_upd0, b2(b_upd0),
      W_msg1, b2(b_msg1), W_upd1, b2(b_upd1),
      W_out, b2(b_out))


# speedup vs baseline: 1.0828x; 1.0828x over previous
import jax
import jax.numpy as jnp
from jax.experimental import pallas as pl
from jax.experimental.pallas import tpu as pltpu

_NORM = 1.0


def _decoder_kernel(a_hbm, h_ref, mask_ref,
                    wm0_ref, bm0_ref, wu0_ref, bu0_ref,
                    wm1_ref, bm1_ref, wu1_ref, bu1_ref,
                    wo_ref, bo_ref, out_ref, m_scr, a_vmem, sems,
                    wc0_scr, bc0_scr, wc1_scr, bc1_scr):
    f32 = jnp.float32
    P = jax.lax.Precision.DEFAULT
    i = pl.program_id(0)
    num_b = pl.num_programs(0) // 2
    b = i // 2
    layer = jax.lax.rem(i, 2)
    slot = jax.lax.rem(b, 2)
    nslot = jax.lax.rem(b + 1, 2)

    n_nodes = out_ref.shape[1]
    n_chunks = 4
    rows = n_nodes // n_chunks

    def _start_copies(src_b, dst_slot):
        for c in range(n_chunks):
            sl = pl.ds(c * rows, rows)
            pltpu.make_async_copy(a_hbm.at[src_b, sl],
                                  a_vmem.at[dst_slot, sl],
                                  sems.at[dst_slot, c]).start()

    @pl.when(i == 0)
    def _():
        _start_copies(0, 0)

    @pl.when((layer == 0) & (b + 1 < num_b))
    def _():
        _start_copies(b + 1, nslot)

    @pl.when(i == 0)
    def _():
        wc0_scr[...] = jnp.dot(wm0_ref[...], wu0_ref[...], precision=P,
                               preferred_element_type=f32)
        bc0_scr[...] = jnp.dot(bm0_ref[...], wu0_ref[...], precision=P,
                               preferred_element_type=f32)
        wc1_scr[...] = jnp.dot(wm1_ref[...], wu1_ref[...], precision=P,
                               preferred_element_type=f32)
        bc1_scr[...] = jnp.dot(bm1_ref[...], wu1_ref[...], precision=P,
                               preferred_element_type=f32)
        m_scr[...] = jnp.dot(h_ref[0], wc0_scr[...], precision=P,
                             preferred_element_type=f32) + bc0_scr[...]

    is_l0 = layer == 0
    bu = jnp.where(is_l0, bu0_ref[...], bu1_ref[...])

    @pl.when(layer == 0)
    def _():
        for c in range(n_chunks):
            sl = pl.ds(c * rows, rows)
            pltpu.make_async_copy(a_hbm.at[b, sl], a_vmem.at[slot, sl],
                                  sems.at[slot, c]).wait()

    agg = jnp.dot(a_vmem[slot], m_scr[...], precision=P,
                  preferred_element_type=f32)
    h_next = jnp.maximum(agg * (1.0 / _NORM) + bu, 0.0)

    @pl.when(layer == 0)
    def _():
        m_scr[...] = jnp.dot(h_next, wc1_scr[...], precision=P,
                             preferred_element_type=f32) + bc1_scr[...]

    @pl.when((layer == 1) & (i + 1 < pl.num_programs(0)))
    def _():
        m_scr[...] = jnp.dot(h_ref[0], wc0_scr[...], precision=P,
                             preferred_element_type=f32) + bc0_scr[...]

    @pl.when(layer == 1)
    def _():
        out = jnp.dot(h_next, wo_ref[...], precision=P,
                      preferred_element_type=f32) + bo_ref[...]
        out_ref[0] = out * mask_ref[0]


def kernel(latent_features, adjacency_matrix, node_mask,
           W_msg0, b_msg0, W_upd0, b_upd0,
           W_msg1, b_msg1, W_upd1, b_upd1,
           W_out, b_out):
    B, N, d_lat = latent_features.shape
    d_hid = W_msg0.shape[1]
    d_out = W_out.shape[1]

    b2 = lambda b: b.reshape(1, -1)

    batch_spec = lambda shape: pl.BlockSpec(shape, lambda i: (i // 2, 0, 0))
    lat_spec = pl.BlockSpec((1, N, d_lat),
                            lambda i: (jnp.minimum((i + 1) // 2, B - 1), 0, 0))
    w_spec = pl.BlockSpec((d_hid, d_hid), lambda i: (0, 0))
    bias_spec = pl.BlockSpec((1, d_hid), lambda i: (0, 0))

    return pl.pallas_call(
        _decoder_kernel,
        grid=(2 * B,),
        in_specs=[
            pl.BlockSpec(memory_space=pltpu.MemorySpace.HBM),
            lat_spec,
            batch_spec((1, N, 1)),
            pl.BlockSpec((d_lat, d_hid), lambda i: (0, 0)), bias_spec,
            w_spec, bias_spec,
            w_spec, bias_spec,
            w_spec, bias_spec,
            pl.BlockSpec((d_hid, d_out), lambda i: (0, 0)),
            pl.BlockSpec((1, d_out), lambda i: (0, 0)),
        ],
        out_specs=batch_spec((1, N, d_out)),
        out_shape=jax.ShapeDtypeStruct((B, N, d_out), jnp.float32),
        scratch_shapes=[
            pltpu.VMEM((N, d_hid), jnp.float32),
            pltpu.VMEM((2, N, N), jnp.float32),
            pltpu.SemaphoreType.DMA((2, 4)),
            pltpu.VMEM((d_hid, d_hid), jnp.float32),
            pltpu.VMEM((1, d_hid), jnp.float32),
            pltpu.VMEM((d_hid, d_hid), jnp.float32),
            pltpu.VMEM((1, d_hid), jnp.float32),
        ],
        compiler_params=pltpu.CompilerParams(
            dimension_semantics=("arbitrary",),
            vmem_limit_bytes=64 * 1024 * 1024,
        ),
    )(adjacency_matrix, latent_features, node_mask,
      W_msg0, b2(b_msg0), W_upd0, b2(b_upd0),
      W_msg1, b2(b_msg1), W_upd1, b2(b_upd1),
      W_out, b2(b_out))


# speedup vs baseline: 1.0856x; 1.0026x over previous
import jax
import jax.numpy as jnp
from jax.experimental import pallas as pl
from jax.experimental.pallas import tpu as pltpu

_NORM = 1.0


def _decoder_kernel(a_hbm, h_ref, mask_ref,
                    wm0_ref, bm0_ref, wu0_ref, bu0_ref,
                    wm1_ref, bm1_ref, wu1_ref, bu1_ref,
                    wo_ref, bo_ref, out_ref, m_scr, a_vmem, sems,
                    wc0_scr, bc0_scr, wc1_scr, bc1_scr):
    f32 = jnp.float32
    P = jax.lax.Precision.DEFAULT
    i = pl.program_id(0)
    num_b = pl.num_programs(0) // 2
    b = i // 2
    layer = jax.lax.rem(i, 2)
    slot = jax.lax.rem(b, 2)
    nslot = jax.lax.rem(b + 1, 2)

    n_nodes = out_ref.shape[1]
    n_chunks = 4
    rows = n_nodes // n_chunks

    def _start_copies(src_b, dst_slot):
        for c in range(n_chunks):
            sl = pl.ds(c * rows, rows)
            pltpu.make_async_copy(a_hbm.at[src_b, sl],
                                  a_vmem.at[dst_slot, sl],
                                  sems.at[dst_slot, c]).start()

    @pl.when(i == 0)
    def _():
        _start_copies(0, 0)

    @pl.when((layer == 0) & (b + 1 < num_b))
    def _():
        _start_copies(b + 1, nslot)

    PH = jax.lax.Precision.HIGHEST

    @pl.when(i == 0)
    def _():
        wc0_scr[...] = jnp.dot(wm0_ref[...], wu0_ref[...], precision=PH,
                               preferred_element_type=f32)
        bc0_scr[...] = jnp.dot(bm0_ref[...], wu0_ref[...], precision=PH,
                               preferred_element_type=f32)
        wc1_scr[...] = jnp.dot(wm1_ref[...], wu1_ref[...], precision=PH,
                               preferred_element_type=f32)
        bc1_scr[...] = jnp.dot(bm1_ref[...], wu1_ref[...], precision=PH,
                               preferred_element_type=f32)
        m_scr[...] = jnp.dot(h_ref[0], wc0_scr[...], precision=P,
                             preferred_element_type=f32) + bc0_scr[...]

    is_l0 = layer == 0
    bu = jnp.where(is_l0, bu0_ref[...], bu1_ref[...])

    @pl.when(layer == 0)
    def _():
        for c in range(n_chunks):
            sl = pl.ds(c * rows, rows)
            pltpu.make_async_copy(a_hbm.at[b, sl], a_vmem.at[slot, sl],
                                  sems.at[slot, c]).wait()

    agg = jnp.dot(a_vmem[slot], m_scr[...], precision=P,
                  preferred_element_type=f32)
    h_next = jnp.maximum(agg * (1.0 / _NORM) + bu, 0.0)

    @pl.when(layer == 0)
    def _():
        m_scr[...] = jnp.dot(h_next, wc1_scr[...], precision=P,
                             preferred_element_type=f32) + bc1_scr[...]

    @pl.when((layer == 1) & (i + 1 < pl.num_programs(0)))
    def _():
        m_scr[...] = jnp.dot(h_ref[0], wc0_scr[...], precision=P,
                             preferred_element_type=f32) + bc0_scr[...]

    @pl.when(layer == 1)
    def _():
        out = jnp.dot(h_next, wo_ref[...], precision=P,
                      preferred_element_type=f32) + bo_ref[...]
        out_ref[0] = out * mask_ref[0]


def kernel(latent_features, adjacency_matrix, node_mask,
           W_msg0, b_msg0, W_upd0, b_upd0,
           W_msg1, b_msg1, W_upd1, b_upd1,
           W_out, b_out):
    B, N, d_lat = latent_features.shape
    d_hid = W_msg0.shape[1]
    d_out = W_out.shape[1]

    b2 = lambda b: b.reshape(1, -1)

    batch_spec = lambda shape: pl.BlockSpec(shape, lambda i: (i // 2, 0, 0))
    lat_spec = pl.BlockSpec((1, N, d_lat),
                            lambda i: (jnp.minimum((i + 1) // 2, B - 1), 0, 0))
    w_spec = pl.BlockSpec((d_hid, d_hid), lambda i: (0, 0))
    bias_spec = pl.BlockSpec((1, d_hid), lambda i: (0, 0))

    return pl.pallas_call(
        _decoder_kernel,
        grid=(2 * B,),
        in_specs=[
            pl.BlockSpec(memory_space=pltpu.MemorySpace.HBM),
            lat_spec,
            batch_spec((1, N, 1)),
            pl.BlockSpec((d_lat, d_hid), lambda i: (0, 0)), bias_spec,
            w_spec, bias_spec,
            w_spec, bias_spec,
            w_spec, bias_spec,
            pl.BlockSpec((d_hid, d_out), lambda i: (0, 0)),
            pl.BlockSpec((1, d_out), lambda i: (0, 0)),
        ],
        out_specs=batch_spec((1, N, d_out)),
        out_shape=jax.ShapeDtypeStruct((B, N, d_out), jnp.float32),
        scratch_shapes=[
            pltpu.VMEM((N, d_hid), jnp.float32),
            pltpu.VMEM((2, N, N), jnp.float32),
            pltpu.SemaphoreType.DMA((2, 4)),
            pltpu.VMEM((d_hid, d_hid), jnp.float32),
            pltpu.VMEM((1, d_hid), jnp.float32),
            pltpu.VMEM((d_hid, d_hid), jnp.float32),
            pltpu.VMEM((1, d_hid), jnp.float32),
        ],
        compiler_params=pltpu.CompilerParams(
            dimension_semantics=("arbitrary",),
            vmem_limit_bytes=64 * 1024 * 1024,
        ),
    )(adjacency_matrix, latent_features, node_mask,
      W_msg0, b2(b_msg0), W_upd0, b2(b_upd0),
      W_msg1, b2(b_msg1), W_upd1, b2(b_upd1),
      W_out, b2(b_out))
